# Initial kernel scaffold; baseline (speedup 1.0000x reference)
import jax
import jax.numpy as jnp
from jax.experimental import pallas as pl


def kernel(rna_embeddings, ss_emb, W_fuse, b_fuse, ln1_g, ln1_b, W_gat, att_src, att_dst, b_gat, gn_w, gn_b, gn_ms, W_gate, b_gate, W_head, b_head, ln2_g, ln2_b, W_fc1, b_fc1, ln3_g, ln3_b, W_fc2, b_fc2, edge_index, batch):
    raise NotImplementedError("write your pallas kernel here")


# speedup vs baseline: 27.9942x; 27.9942x over previous
import functools

import jax
import jax.numpy as jnp
from jax import lax
from jax.experimental import pallas as pl
from jax.experimental.pallas import tpu as pltpu
from jax.experimental.pallas import tpu_sc as plsc

N = 10000
H = 128
G = 64
EL = 650000
R = 400
GRID = N // R

NC = 2
NS = 16
C = 128
NCH = 159
EPW = NCH * C
E_HALF = EPW * NS
EPAD = E_HALF * NC
NPAD = 10240
RPT = NPAD // NS


def _rcp(x):
    r = 1.0 / x
    return r * (2.0 - x * r)


def _rsqrt(x):
    r = lax.rsqrt(x)
    return r * (1.5 - 0.5 * x * r * r)


def _ln(x, g, b):
    m = jnp.mean(x, axis=-1, keepdims=True)
    v = jnp.mean((x - m) ** 2, axis=-1, keepdims=True)
    return (x - m) * _rsqrt(v + 1e-5) * g + b


def _fuse_body(rna_ref, ss_ref, w1_ref, w2_ref, b_ref, g_ref, bb_ref, out_ref):
    x = jnp.dot(rna_ref[...], w1_ref[...], preferred_element_type=jnp.float32, precision=lax.Precision.HIGHEST)
    x = x + jnp.dot(ss_ref[...], w2_ref[...], preferred_element_type=jnp.float32, precision=lax.Precision.HIGHEST)
    x = x + b_ref[...]
    out_ref[...] = jnp.maximum(_ln(x, g_ref[...], bb_ref[...]), 0.0)


def _fuse(rna, ss, W_fuse, b_fuse, g, b):
    w1 = W_fuse[:, :rna.shape[1]].T
    w2 = W_fuse[:, rna.shape[1]:].T
    return pl.pallas_call(
        _fuse_body,
        grid=(GRID,),
        in_specs=[
            pl.BlockSpec((R, rna.shape[1]), lambda i: (i, 0)),
            pl.BlockSpec((R, ss.shape[1]), lambda i: (i, 0)),
            pl.BlockSpec(w1.shape, lambda i: (0, 0)),
            pl.BlockSpec(w2.shape, lambda i: (0, 0)),
            pl.BlockSpec((1, H), lambda i: (0, 0)),
            pl.BlockSpec((1, H), lambda i: (0, 0)),
            pl.BlockSpec((1, H), lambda i: (0, 0)),
        ],
        out_specs=pl.BlockSpec((R, H), lambda i: (i, 0)),
        out_shape=jax.ShapeDtypeStruct((N, H), jnp.float32),
    )(rna, ss, w1, w2, b_fuse.reshape(1, H), g.reshape(1, H), b.reshape(1, H))


def _gatpre_body(h_ref, w_ref, as_ref, ad_ref, hw_ref, s_ref, d_ref, ms_ref, md_ref):
    hw = jnp.dot(h_ref[...], w_ref[...], preferred_element_type=jnp.float32, precision=lax.Precision.HIGHEST)
    hw_ref[...] = hw
    s = jnp.dot(hw, as_ref[...], preferred_element_type=jnp.float32, precision=lax.Precision.HIGHEST)
    d = jnp.dot(hw, ad_ref[...], preferred_element_type=jnp.float32, precision=lax.Precision.HIGHEST)
    s_ref[...] = s
    d_ref[...] = d
    i = pl.program_id(0)

    @pl.when(i == 0)
    def _():
        ms_ref[...] = jnp.full((1, 1), -1e30, jnp.float32)
        md_ref[...] = jnp.full((1, 1), -1e30, jnp.float32)

    ms_ref[...] = jnp.maximum(ms_ref[...], jnp.max(s).reshape(1, 1))
    md_ref[...] = jnp.maximum(md_ref[...], jnp.max(d).reshape(1, 1))


def _gatpre(x, W_gat, att_src, att_dst):
    return pl.pallas_call(
        _gatpre_body,
        grid=(GRID,),
        in_specs=[
            pl.BlockSpec((R, H), lambda i: (i, 0)),
            pl.BlockSpec((H, H), lambda i: (0, 0)),
            pl.BlockSpec((H, 1), lambda i: (0, 0)),
            pl.BlockSpec((H, 1), lambda i: (0, 0)),
        ],
        out_specs=[
            pl.BlockSpec((R, H), lambda i: (i, 0)),
            pl.BlockSpec((R, 1), lambda i: (i, 0)),
            pl.BlockSpec((R, 1), lambda i: (i, 0)),
            pl.BlockSpec((1, 1), lambda i: (0, 0)),
            pl.BlockSpec((1, 1), lambda i: (0, 0)),
        ],
        out_shape=[
            jax.ShapeDtypeStruct((N, H), jnp.float32),
            jax.ShapeDtypeStruct((N, 1), jnp.float32),
            jax.ShapeDtypeStruct((N, 1), jnp.float32),
            jax.ShapeDtypeStruct((1, 1), jnp.float32),
            jax.ShapeDtypeStruct((1, 1), jnp.float32),
        ],
    )(x, W_gat.T, att_src.reshape(H, 1), att_dst.reshape(H, 1))


def _gat_agg_body(hw_hbm, asrc_hbm, adst_hbm, src_hbm, dst_hbm, m_hbm,
                  acc_out, den_out,
                  asrc_v, adst_v, srcc_v, dstc_v, ea_v, rows_v,
                  m_v, acc_sh, den_sh, sem):
    cid = lax.axis_index("c")
    sid = lax.axis_index("s")
    zero16 = jnp.zeros((16,), jnp.float32)

    def _zb(r, carry):
        for kk in range(H // 16):
            rows_v[r, pl.ds(kk * 16, 16)] = zero16
        return carry
    lax.fori_loop(0, C, _zb, 0)
    for kk in range(C // 16):
        ea_v[pl.ds(kk * 16, 16)] = zero16
    for j in range(RPT // C):
        pltpu.sync_copy(rows_v, acc_sh.at[pl.ds(sid * RPT + j * C, C)])
        pltpu.sync_copy(ea_v, den_sh.at[pl.ds(sid * RPT + j * C, C)])

    pltpu.sync_copy(asrc_hbm, asrc_v)
    pltpu.sync_copy(adst_hbm, adst_v)
    pltpu.sync_copy(m_hbm, m_v)
    ebase = cid * E_HALF + sid * EPW
    plsc.subcore_barrier()
    mvec = m_v[...]

    def _chunk(i, carry):
        eb = ebase + i * C
        pltpu.sync_copy(src_hbm.at[pl.ds(eb, C)], srcc_v)
        pltpu.sync_copy(dst_hbm.at[pl.ds(eb, C)], dstc_v)
        gat = pltpu.async_copy(hw_hbm.at[srcc_v], rows_v, sem)
        for kk in range(C // 16):
            si = srcc_v[pl.ds(kk * 16, 16)]
            di = dstc_v[pl.ds(kk * 16, 16)]
            a = plsc.load_gather(asrc_v, [si]) + plsc.load_gather(adst_v, [di])
            a = jnp.where(a > 0.0, a, 0.2 * a)
            e = jnp.exp(a - mvec)
            eid = eb + kk * 16 + lax.iota(jnp.int32, 16)
            e = jnp.where(eid < EL, e, 0.0)
            ea_v[pl.ds(kk * 16, 16)] = e
        gat.wait()

        def _scale(j, c2):
            e16 = ea_v[pl.ds(j * 16, 16)]
            for t in range(16):
                r = j * 16 + t
                c0 = e16[t]
                for kk in range(H // 16):
                    rows_v[r, pl.ds(kk * 16, 16)] = rows_v[r, pl.ds(kk * 16, 16)] * c0
            return c2
        lax.fori_loop(0, C // 16, _scale, 0)
        pltpu.sync_copy(rows_v, acc_sh.at[dstc_v], add=True)
        pltpu.sync_copy(ea_v, den_sh.at[dstc_v], add=True)
        return carry

    lax.fori_loop(0, NCH, _chunk, 0)
    plsc.subcore_barrier()
    for j in range(RPT // C):
        pltpu.sync_copy(acc_sh.at[pl.ds(sid * RPT + j * C, C)],
                        acc_out.at[cid, pl.ds(sid * RPT + j * C, C)])
    pltpu.sync_copy(den_sh.at[pl.ds(sid * RPT, RPT)],
                    den_out.at[cid, pl.ds(sid * RPT, RPT)])


def _gat_agg(hw, asrc, adst, srcp, dstp, m16):
    mesh = plsc.VectorSubcoreMesh(core_axis_name="c", subcore_axis_name="s",
                                  num_cores=NC, num_subcores=NS)
    kfn = pl.kernel(
        _gat_agg_body,
        out_type=(jax.ShapeDtypeStruct((NC, NPAD, H), jnp.float32),
                  jax.ShapeDtypeStruct((NC, NPAD), jnp.float32)),
        mesh=mesh,
        compiler_params=pltpu.CompilerParams(needs_layout_passes=False),
        scratch_types=[
            pltpu.VMEM((N,), jnp.float32),
            pltpu.VMEM((N,), jnp.float32),
            pltpu.VMEM((C,), jnp.int32),
            pltpu.VMEM((C,), jnp.int32),
            pltpu.VMEM((C,), jnp.float32),
            pltpu.VMEM((C, H), jnp.float32),
            pltpu.VMEM((16,), jnp.float32),
            pltpu.VMEM_SHARED((NPAD, H), jnp.float32),
            pltpu.VMEM_SHARED((NPAD,), jnp.float32),
            pltpu.SemaphoreType.DMA,
        ],
    )
    return kfn(hw, asrc, adst, srcp, dstp, m16)


def _comb_body(acc_ref, den_ref, b_ref, batch_ref, gat_ref, s1_ref, cnt_ref):
    a = acc_ref[0] + acc_ref[1]
    den = den_ref[0] + den_ref[1]
    gat = a * _rcp(den) + b_ref[...]
    gat_ref[...] = gat
    oh = (batch_ref[...] == lax.broadcasted_iota(jnp.int32, (1, G), 1)
          ).astype(jnp.float32)
    i = pl.program_id(0)

    @pl.when(i == 0)
    def _():
        s1_ref[...] = jnp.zeros_like(s1_ref)
        cnt_ref[...] = jnp.zeros_like(cnt_ref)

    dn = (((0,), (0,)), ((), ()))
    s1_ref[...] += lax.dot_general(oh, gat, dn, preferred_element_type=jnp.float32, precision=lax.Precision.HIGHEST)
    cnt_ref[...] += jnp.sum(oh, axis=0, keepdims=True).T


def _combstats(acc, den3, b_gat, batch2):
    return pl.pallas_call(
        _comb_body,
        grid=(GRID,),
        in_specs=[
            pl.BlockSpec((NC, R, H), lambda i: (0, i, 0)),
            pl.BlockSpec((NC, R, 1), lambda i: (0, i, 0)),
            pl.BlockSpec((1, H), lambda i: (0, 0)),
            pl.BlockSpec((R, 1), lambda i: (i, 0)),
        ],
        out_specs=[
            pl.BlockSpec((R, H), lambda i: (i, 0)),
            pl.BlockSpec((G, H), lambda i: (0, 0)),
            pl.BlockSpec((G, 1), lambda i: (0, 0)),
        ],
        out_shape=[
            jax.ShapeDtypeStruct((N, H), jnp.float32),
            jax.ShapeDtypeStruct((G, H), jnp.float32),
            jax.ShapeDtypeStruct((G, 1), jnp.float32),
        ],
    )(acc, den3, b_gat.reshape(1, H), batch2)


def _var_body(x_ref, s1_ref, cnt_ref, ms_ref, batch_ref, v_ref, mean_ref):
    i = pl.program_id(0)

    @pl.when(i == 0)
    def _():
        cnt = jnp.maximum(cnt_ref[...], 1.0)
        mean_ref[...] = s1_ref[...] / cnt * ms_ref[...]
        v_ref[...] = jnp.zeros_like(v_ref)

    oh = (batch_ref[...] == lax.broadcasted_iota(jnp.int32, (1, G), 1)
          ).astype(jnp.float32)
    mm = jnp.dot(oh, mean_ref[...], preferred_element_type=jnp.float32, precision=lax.Precision.HIGHEST)
    d = x_ref[...] - mm
    dn = (((0,), (0,)), ((), ()))
    v_ref[...] += lax.dot_general(oh, d * d, dn, preferred_element_type=jnp.float32, precision=lax.Precision.HIGHEST)


def _varstats(x, s1, cnt, gn_ms, batch2):
    return pl.pallas_call(
        _var_body,
        grid=(GRID,),
        in_specs=[
            pl.BlockSpec((R, H), lambda i: (i, 0)),
            pl.BlockSpec((G, H), lambda i: (0, 0)),
            pl.BlockSpec((G, 1), lambda i: (0, 0)),
            pl.BlockSpec((1, H), lambda i: (0, 0)),
            pl.BlockSpec((R, 1), lambda i: (i, 0)),
        ],
        out_specs=[
            pl.BlockSpec((G, H), lambda i: (0, 0)),
            pl.BlockSpec((G, H), lambda i: (0, 0)),
        ],
        out_shape=[
            jax.ShapeDtypeStruct((G, H), jnp.float32),
            jax.ShapeDtypeStruct((G, H), jnp.float32),
        ],
    )(x, s1, cnt, gn_ms.reshape(1, H), batch2)[0]


def _gn_body(has_res, x_ref, s1_ref, v_ref, cnt_ref, w_ref, b_ref, ms_ref,
             batch_ref, *rest):
    if has_res:
        res_ref, out_ref, scale_ref, shift_ref = rest
    else:
        out_ref, scale_ref, shift_ref = rest
    i = pl.program_id(0)

    @pl.when(i == 0)
    def _():
        cnt = jnp.maximum(cnt_ref[...], 1.0)
        mm = s1_ref[...] / cnt * ms_ref[...]
        var = v_ref[...] / cnt
        rstd = _rsqrt(var + 1e-5)
        scale = w_ref[...] * rstd
        scale_ref[...] = scale
        shift_ref[...] = b_ref[...] - scale * mm

    oh = (batch_ref[...] == lax.broadcasted_iota(jnp.int32, (1, G), 1)
          ).astype(jnp.float32)
    sc = jnp.dot(oh, scale_ref[...], preferred_element_type=jnp.float32, precision=lax.Precision.HIGHEST)
    sh = jnp.dot(oh, shift_ref[...], preferred_element_type=jnp.float32, precision=lax.Precision.HIGHEST)
    y = jnp.maximum(x_ref[...] * sc + sh, 0.0)
    if has_res:
        y = y + res_ref[...]
    out_ref[...] = y


def _gn_apply(x, s1, v, cnt, gn_w, gn_b, gn_ms, batch2, res=None):
    has_res = res is not None
    in_specs = [
        pl.BlockSpec((R, H), lambda i: (i, 0)),
        pl.BlockSpec((G, H), lambda i: (0, 0)),
        pl.BlockSpec((G, H), lambda i: (0, 0)),
        pl.BlockSpec((G, 1), lambda i: (0, 0)),
        pl.BlockSpec((1, H), lambda i: (0, 0)),
        pl.BlockSpec((1, H), lambda i: (0, 0)),
        pl.BlockSpec((1, H), lambda i: (0, 0)),
        pl.BlockSpec((R, 1), lambda i: (i, 0)),
    ]
    args = [x, s1, v, cnt, gn_w.reshape(1, H), gn_b.reshape(1, H),
            gn_ms.reshape(1, H), batch2]
    if has_res:
        in_specs.append(pl.BlockSpec((R, H), lambda i: (i, 0)))
        args.append(res)
    return pl.pallas_call(
        functools.partial(_gn_body, has_res),
        grid=(GRID,),
        in_specs=in_specs,
        out_specs=[
            pl.BlockSpec((R, H), lambda i: (i, 0)),
            pl.BlockSpec((G, H), lambda i: (0, 0)),
            pl.BlockSpec((G, H), lambda i: (0, 0)),
        ],
        out_shape=[
            jax.ShapeDtypeStruct((N, H), jnp.float32),
            jax.ShapeDtypeStruct((G, H), jnp.float32),
            jax.ShapeDtypeStruct((G, H), jnp.float32),
        ],
    )(*args)[0]


def _final_body(h1_ref, h2_ref, wg1_ref, wg2_ref, bg_ref, wh_ref, bh_ref,
                g2_ref, b2_ref, wf1_ref, bf1_ref, g3_ref, b3_ref, wf2_ref,
                bf2_ref, out_ref):
    h1 = h1_ref[...]
    h2 = h2_ref[...]
    z = (jnp.dot(h1, wg1_ref[...], preferred_element_type=jnp.float32, precision=lax.Precision.HIGHEST)
         + jnp.dot(h2, wg2_ref[...], preferred_element_type=jnp.float32, precision=lax.Precision.HIGHEST)
         + bg_ref[...])
    gate = _rcp(1.0 + jnp.exp(-z))
    h = gate * h1 + (1.0 - gate) * h2
    y = jnp.dot(h, wh_ref[...], preferred_element_type=jnp.float32, precision=lax.Precision.HIGHEST) + bh_ref[...]
    y = jnp.maximum(_ln(y, g2_ref[...], b2_ref[...]), 0.0)
    y = jnp.dot(y, wf1_ref[...], preferred_element_type=jnp.float32, precision=lax.Precision.HIGHEST) + bf1_ref[...]
    y = jnp.maximum(_ln(y, g3_ref[...], b3_ref[...]), 0.0)
    out_ref[...] = jnp.dot(y, wf2_ref[...], preferred_element_type=jnp.float32, precision=lax.Precision.HIGHEST) + bf2_ref[...]


def _final(h1, h2, W_gate, b_gate, W_head, b_head, ln2_g, ln2_b, W_fc1, b_fc1,
           ln3_g, ln3_b, W_fc2, b_fc2):
    H2 = H // 2
    full = lambda shape: pl.BlockSpec(shape, lambda i: (0, 0))
    return pl.pallas_call(
        _final_body,
        grid=(GRID,),
        in_specs=[
            pl.BlockSpec((R, H), lambda i: (i, 0)),
            pl.BlockSpec((R, H), lambda i: (i, 0)),
            full((H, H)), full((H, H)), full((1, H)),
            full((H, H)), full((1, H)),
            full((1, H)), full((1, H)),
            full((H, H2)), full((1, H2)),
            full((1, H2)), full((1, H2)),
            full((H2, 1)), full((1, 1)),
        ],
        out_specs=pl.BlockSpec((R, 1), lambda i: (i, 0)),
        out_shape=jax.ShapeDtypeStruct((N, 1), jnp.float32),
    )(h1, h2, W_gate[:, :H].T, W_gate[:, H:].T, b_gate.reshape(1, H),
      W_head.T, b_head.reshape(1, H), ln2_g.reshape(1, H), ln2_b.reshape(1, H),
      W_fc1.T, b_fc1.reshape(1, H2), ln3_g.reshape(1, H2), ln3_b.reshape(1, H2),
      W_fc2.T, b_fc2.reshape(1, 1))


def kernel(rna_embeddings, ss_emb, W_fuse, b_fuse, ln1_g, ln1_b, W_gat,
           att_src, att_dst, b_gat, gn_w, gn_b, gn_ms, W_gate, b_gate, W_head,
           b_head, ln2_g, ln2_b, W_fc1, b_fc1, ln3_g, ln3_b, W_fc2, b_fc2,
           edge_index, batch):
    loop = jnp.arange(N, dtype=edge_index.dtype)
    srcp = jnp.pad(jnp.concatenate([edge_index[0], loop]), (0, EPAD - EL))
    dstp = jnp.pad(jnp.concatenate([edge_index[1], loop]), (0, EPAD - EL))
    batch2 = batch.reshape(N, 1)

    h = _fuse(rna_embeddings, ss_emb, W_fuse, b_fuse, ln1_g, ln1_b)

    def gat_block(x, res):
        hw, s, d, ms, md = _gatpre(x, W_gat, att_src, att_dst)
        m = ms[0, 0] + md[0, 0]
        m = jnp.where(m > 0.0, m, 0.2 * m)
        m16 = jnp.full((16,), m, jnp.float32)
        acc, den = _gat_agg(hw, s.reshape(N), d.reshape(N), srcp, dstp, m16)
        gat, s1, cnt = _combstats(acc, den.reshape(NC, NPAD, 1), b_gat, batch2)
        v = _varstats(gat, s1, cnt, gn_ms, batch2)
        return _gn_apply(gat, s1, v, cnt, gn_w, gn_b, gn_ms, batch2, res=res)

    h1 = gat_block(h, None)
    h2 = gat_block(h1, h)
    out = _final(h1, h2, W_gate, b_gate, W_head, b_head, ln2_g, ln2_b,
                 W_fc1, b_fc1, ln3_g, ln3_b, W_fc2, b_fc2)
    return out.reshape(N)


# speedup vs baseline: 37.6617x; 1.3453x over previous
import functools

import jax
import jax.numpy as jnp
from jax import lax
from jax.experimental import pallas as pl
from jax.experimental.pallas import tpu as pltpu
from jax.experimental.pallas import tpu_sc as plsc

N = 10000
H = 128
G = 64
EL = 650000
R = 400
GRID = N // R

NC = 2
NS = 16
C = 96
NCH = 212
EPW = NCH * C
E_HALF = EPW * NS
EPAD = E_HALF * NC
NPAD = 10240
RPT = NPAD // NS


def _rcp(x):
    r = 1.0 / x
    return r * (2.0 - x * r)


def _rsqrt(x):
    r = lax.rsqrt(x)
    return r * (1.5 - 0.5 * x * r * r)


def _ln(x, g, b):
    m = jnp.mean(x, axis=-1, keepdims=True)
    v = jnp.mean((x - m) ** 2, axis=-1, keepdims=True)
    return (x - m) * _rsqrt(v + 1e-5) * g + b


def _fuse_body(rna_ref, ss_ref, w1_ref, w2_ref, b_ref, g_ref, bb_ref, out_ref):
    x = jnp.dot(rna_ref[...], w1_ref[...], preferred_element_type=jnp.float32, precision=lax.Precision.HIGHEST)
    x = x + jnp.dot(ss_ref[...], w2_ref[...], preferred_element_type=jnp.float32, precision=lax.Precision.HIGHEST)
    x = x + b_ref[...]
    out_ref[...] = jnp.maximum(_ln(x, g_ref[...], bb_ref[...]), 0.0)


def _fuse(rna, ss, W_fuse, b_fuse, g, b):
    w1 = W_fuse[:, :rna.shape[1]].T
    w2 = W_fuse[:, rna.shape[1]:].T
    return pl.pallas_call(
        _fuse_body,
        grid=(GRID,),
        in_specs=[
            pl.BlockSpec((R, rna.shape[1]), lambda i: (i, 0)),
            pl.BlockSpec((R, ss.shape[1]), lambda i: (i, 0)),
            pl.BlockSpec(w1.shape, lambda i: (0, 0)),
            pl.BlockSpec(w2.shape, lambda i: (0, 0)),
            pl.BlockSpec((1, H), lambda i: (0, 0)),
            pl.BlockSpec((1, H), lambda i: (0, 0)),
            pl.BlockSpec((1, H), lambda i: (0, 0)),
        ],
        out_specs=pl.BlockSpec((R, H), lambda i: (i, 0)),
        out_shape=jax.ShapeDtypeStruct((N, H), jnp.float32),
    )(rna, ss, w1, w2, b_fuse.reshape(1, H), g.reshape(1, H), b.reshape(1, H))


def _gatpre_body(h_ref, w_ref, as_ref, ad_ref, hw_ref, s_ref, d_ref, ms_ref, md_ref):
    hw = jnp.dot(h_ref[...], w_ref[...], preferred_element_type=jnp.float32, precision=lax.Precision.HIGHEST)
    hw_ref[...] = hw
    s = jnp.dot(hw, as_ref[...], preferred_element_type=jnp.float32, precision=lax.Precision.HIGHEST)
    d = jnp.dot(hw, ad_ref[...], preferred_element_type=jnp.float32, precision=lax.Precision.HIGHEST)
    s_ref[...] = s
    d_ref[...] = d
    i = pl.program_id(0)

    @pl.when(i == 0)
    def _():
        ms_ref[...] = jnp.full((1, 1), -1e30, jnp.float32)
        md_ref[...] = jnp.full((1, 1), -1e30, jnp.float32)

    ms_ref[...] = jnp.maximum(ms_ref[...], jnp.max(s).reshape(1, 1))
    md_ref[...] = jnp.maximum(md_ref[...], jnp.max(d).reshape(1, 1))


def _gatpre(x, W_gat, att_src, att_dst):
    return pl.pallas_call(
        _gatpre_body,
        grid=(GRID,),
        in_specs=[
            pl.BlockSpec((R, H), lambda i: (i, 0)),
            pl.BlockSpec((H, H), lambda i: (0, 0)),
            pl.BlockSpec((H, 1), lambda i: (0, 0)),
            pl.BlockSpec((H, 1), lambda i: (0, 0)),
        ],
        out_specs=[
            pl.BlockSpec((R, H), lambda i: (i, 0)),
            pl.BlockSpec((R, 1), lambda i: (i, 0)),
            pl.BlockSpec((R, 1), lambda i: (i, 0)),
            pl.BlockSpec((1, 1), lambda i: (0, 0)),
            pl.BlockSpec((1, 1), lambda i: (0, 0)),
        ],
        out_shape=[
            jax.ShapeDtypeStruct((N, H), jnp.float32),
            jax.ShapeDtypeStruct((N, 1), jnp.float32),
            jax.ShapeDtypeStruct((N, 1), jnp.float32),
            jax.ShapeDtypeStruct((1, 1), jnp.float32),
            jax.ShapeDtypeStruct((1, 1), jnp.float32),
        ],
    )(x, W_gat.T, att_src.reshape(H, 1), att_dst.reshape(H, 1))


def _gat_agg_body(hw_hbm, asrc_hbm, adst_hbm, src_hbm, dst_hbm, m_hbm,
                  acc_out, den_out,
                  asrc_v, adst_v, srcc_v, dstc_v, dsts_v, ea_v, eas_v, rows_v,
                  m_v, acc_sh, den_sh,
                  sis0, sis1, sid0, sid1, sg0, sg1, ssc0, ssc1, sd0, sd1):
    cid = lax.axis_index("c")
    sid = lax.axis_index("s")
    si_s = (sis0, sis1)
    si_d = (sid0, sid1)
    sg = (sg0, sg1)
    ssc = (ssc0, ssc1)
    sd = (sd0, sd1)
    zero16 = jnp.zeros((16,), jnp.float32)

    def _zb(r, carry):
        for kk in range(H // 16):
            rows_v[0, r, pl.ds(kk * 16, 16)] = zero16
        return carry
    lax.fori_loop(0, 64, _zb, 0)
    for kk in range(C // 16):
        ea_v[0, pl.ds(kk * 16, 16)] = zero16
    for j in range(RPT // 64):
        pltpu.sync_copy(rows_v.at[0, pl.ds(0, 64)],
                        acc_sh.at[pl.ds(sid * RPT + j * 64, 64)])
        pltpu.sync_copy(ea_v.at[0, pl.ds(0, 64)],
                        den_sh.at[pl.ds(sid * RPT + j * 64, 64)])

    pltpu.sync_copy(asrc_hbm, asrc_v)
    pltpu.sync_copy(adst_hbm, adst_v)
    pltpu.sync_copy(m_hbm, m_v)
    ebase = cid * E_HALF + sid * EPW
    plsc.subcore_barrier()
    mvec = m_v[...]

    pltpu.async_copy(src_hbm.at[pl.ds(ebase, C)], srcc_v.at[0], si_s[0])
    pltpu.async_copy(dst_hbm.at[pl.ds(ebase, C)], dstc_v.at[0], si_d[0])

    def _do_chunk(i2, b, eb):
        pltpu.make_async_copy(src_hbm.at[pl.ds(eb, C)], srcc_v.at[b], si_s[b]).wait()
        pltpu.make_async_copy(dst_hbm.at[pl.ds(eb, C)], dstc_v.at[b], si_d[b]).wait()
        b1 = 1 - b
        pltpu.async_copy(src_hbm.at[pl.ds(eb + C, C)], srcc_v.at[b1], si_s[b1])
        pltpu.async_copy(dst_hbm.at[pl.ds(eb + C, C)], dstc_v.at[b1], si_d[b1])

        @pl.when(i2 > 0)
        def _():
            pltpu.make_async_copy(rows_v.at[b], acc_sh.at[dsts_v.at[b]], ssc[b]).wait()
            pltpu.make_async_copy(eas_v.at[b], den_sh.at[dsts_v.at[b]], sd[b]).wait()

        gath = pltpu.async_copy(hw_hbm.at[srcc_v.at[b]], rows_v.at[b], sg[b])
        for kk in range(C // 16):
            si = srcc_v[b, pl.ds(kk * 16, 16)]
            di = dstc_v[b, pl.ds(kk * 16, 16)]
            a = plsc.load_gather(asrc_v, [si]) + plsc.load_gather(adst_v, [di])
            a = jnp.where(a > 0.0, a, 0.2 * a)
            e = jnp.exp(a - mvec)
            eid = eb + kk * 16 + lax.iota(jnp.int32, 16)
            e = jnp.where(eid < EL, e, 0.0)
            ea_v[b, pl.ds(kk * 16, 16)] = e
        gath.wait()

        def _scale(jj, c2):
            e16 = ea_v[b, pl.ds(jj * 16, 16)]
            for t in range(16):
                r = jj * 16 + t
                c0 = e16[t]
                for kk in range(H // 16):
                    rows_v[b, r, pl.ds(kk * 16, 16)] = rows_v[b, r, pl.ds(kk * 16, 16)] * c0
            return c2
        lax.fori_loop(0, C // 16, _scale, 0)
        for kk in range(C // 16):
            dsts_v[b, pl.ds(kk * 16, 16)] = dstc_v[b, pl.ds(kk * 16, 16)]
            eas_v[b, pl.ds(kk * 16, 16)] = ea_v[b, pl.ds(kk * 16, 16)]
        pltpu.async_copy(rows_v.at[b], acc_sh.at[dsts_v.at[b]], ssc[b], add=True)
        pltpu.async_copy(eas_v.at[b], den_sh.at[dsts_v.at[b]], sd[b], add=True)

    def _pair(i2, carry):
        eb = ebase + i2 * (2 * C)
        _do_chunk(i2, 0, eb)
        _do_chunk(i2, 1, eb + C)
        return carry

    lax.fori_loop(0, NCH // 2, _pair, 0)

    for b in range(2):
        pltpu.make_async_copy(rows_v.at[b], acc_sh.at[dsts_v.at[b]], ssc[b]).wait()
        pltpu.make_async_copy(eas_v.at[b], den_sh.at[dsts_v.at[b]], sd[b]).wait()
    pltpu.make_async_copy(src_hbm.at[pl.ds(ebase, C)], srcc_v.at[0], si_s[0]).wait()
    pltpu.make_async_copy(dst_hbm.at[pl.ds(ebase, C)], dstc_v.at[0], si_d[0]).wait()

    plsc.subcore_barrier()
    for j in range(RPT // 64):
        pltpu.sync_copy(acc_sh.at[pl.ds(sid * RPT + j * 64, 64)],
                        acc_out.at[cid, pl.ds(sid * RPT + j * 64, 64)])
    pltpu.sync_copy(den_sh.at[pl.ds(sid * RPT, RPT)],
                    den_out.at[cid, pl.ds(sid * RPT, RPT)])


def _gat_agg(hw, asrc, adst, srcp, dstp, m16):
    mesh = plsc.VectorSubcoreMesh(core_axis_name="c", subcore_axis_name="s",
                                  num_cores=NC, num_subcores=NS)
    kfn = pl.kernel(
        _gat_agg_body,
        out_type=(jax.ShapeDtypeStruct((NC, NPAD, H), jnp.float32),
                  jax.ShapeDtypeStruct((NC, NPAD), jnp.float32)),
        mesh=mesh,
        compiler_params=pltpu.CompilerParams(needs_layout_passes=False),
        scratch_types=[
            pltpu.VMEM((N,), jnp.float32),
            pltpu.VMEM((N,), jnp.float32),
            pltpu.VMEM((2, C), jnp.int32),
            pltpu.VMEM((2, C), jnp.int32),
            pltpu.VMEM((2, C), jnp.int32),
            pltpu.VMEM((2, C), jnp.float32),
            pltpu.VMEM((2, C), jnp.float32),
            pltpu.VMEM((2, C, H), jnp.float32),
            pltpu.VMEM((16,), jnp.float32),
            pltpu.VMEM_SHARED((NPAD, H), jnp.float32),
            pltpu.VMEM_SHARED((NPAD,), jnp.float32),
        ] + [pltpu.SemaphoreType.DMA] * 10,
    )
    return kfn(hw, asrc, adst, srcp, dstp, m16)


def _comb_body(acc_ref, den_ref, b_ref, batch_ref, gat_ref, s1_ref, cnt_ref):
    a = acc_ref[0] + acc_ref[1]
    den = den_ref[0] + den_ref[1]
    gat = a * _rcp(den) + b_ref[...]
    gat_ref[...] = gat
    oh = (batch_ref[...] == lax.broadcasted_iota(jnp.int32, (1, G), 1)
          ).astype(jnp.float32)
    i = pl.program_id(0)

    @pl.when(i == 0)
    def _():
        s1_ref[...] = jnp.zeros_like(s1_ref)
        cnt_ref[...] = jnp.zeros_like(cnt_ref)

    dn = (((0,), (0,)), ((), ()))
    s1_ref[...] += lax.dot_general(oh, gat, dn, preferred_element_type=jnp.float32, precision=lax.Precision.HIGHEST)
    cnt_ref[...] += jnp.sum(oh, axis=0, keepdims=True).T


def _combstats(acc, den3, b_gat, batch2):
    return pl.pallas_call(
        _comb_body,
        grid=(GRID,),
        in_specs=[
            pl.BlockSpec((NC, R, H), lambda i: (0, i, 0)),
            pl.BlockSpec((NC, R, 1), lambda i: (0, i, 0)),
            pl.BlockSpec((1, H), lambda i: (0, 0)),
            pl.BlockSpec((R, 1), lambda i: (i, 0)),
        ],
        out_specs=[
            pl.BlockSpec((R, H), lambda i: (i, 0)),
            pl.BlockSpec((G, H), lambda i: (0, 0)),
            pl.BlockSpec((G, 1), lambda i: (0, 0)),
        ],
        out_shape=[
            jax.ShapeDtypeStruct((N, H), jnp.float32),
            jax.ShapeDtypeStruct((G, H), jnp.float32),
            jax.ShapeDtypeStruct((G, 1), jnp.float32),
        ],
    )(acc, den3, b_gat.reshape(1, H), batch2)


def _var_body(x_ref, s1_ref, cnt_ref, ms_ref, batch_ref, v_ref, mean_ref):
    i = pl.program_id(0)

    @pl.when(i == 0)
    def _():
        cnt = jnp.maximum(cnt_ref[...], 1.0)
        mean_ref[...] = s1_ref[...] / cnt * ms_ref[...]
        v_ref[...] = jnp.zeros_like(v_ref)

    oh = (batch_ref[...] == lax.broadcasted_iota(jnp.int32, (1, G), 1)
          ).astype(jnp.float32)
    mm = jnp.dot(oh, mean_ref[...], preferred_element_type=jnp.float32, precision=lax.Precision.HIGHEST)
    d = x_ref[...] - mm
    dn = (((0,), (0,)), ((), ()))
    v_ref[...] += lax.dot_general(oh, d * d, dn, preferred_element_type=jnp.float32, precision=lax.Precision.HIGHEST)


def _varstats(x, s1, cnt, gn_ms, batch2):
    return pl.pallas_call(
        _var_body,
        grid=(GRID,),
        in_specs=[
            pl.BlockSpec((R, H), lambda i: (i, 0)),
            pl.BlockSpec((G, H), lambda i: (0, 0)),
            pl.BlockSpec((G, 1), lambda i: (0, 0)),
            pl.BlockSpec((1, H), lambda i: (0, 0)),
            pl.BlockSpec((R, 1), lambda i: (i, 0)),
        ],
        out_specs=[
            pl.BlockSpec((G, H), lambda i: (0, 0)),
            pl.BlockSpec((G, H), lambda i: (0, 0)),
        ],
        out_shape=[
            jax.ShapeDtypeStruct((G, H), jnp.float32),
            jax.ShapeDtypeStruct((G, H), jnp.float32),
        ],
    )(x, s1, cnt, gn_ms.reshape(1, H), batch2)[0]


def _gn_body(has_res, x_ref, s1_ref, v_ref, cnt_ref, w_ref, b_ref, ms_ref,
             batch_ref, *rest):
    if has_res:
        res_ref, out_ref, scale_ref, shift_ref = rest
    else:
        out_ref, scale_ref, shift_ref = rest
    i = pl.program_id(0)

    @pl.when(i == 0)
    def _():
        cnt = jnp.maximum(cnt_ref[...], 1.0)
        mm = s1_ref[...] / cnt * ms_ref[...]
        var = v_ref[...] / cnt
        rstd = _rsqrt(var + 1e-5)
        scale = w_ref[...] * rstd
        scale_ref[...] = scale
        shift_ref[...] = b_ref[...] - scale * mm

    oh = (batch_ref[...] == lax.broadcasted_iota(jnp.int32, (1, G), 1)
          ).astype(jnp.float32)
    sc = jnp.dot(oh, scale_ref[...], preferred_element_type=jnp.float32, precision=lax.Precision.HIGHEST)
    sh = jnp.dot(oh, shift_ref[...], preferred_element_type=jnp.float32, precision=lax.Precision.HIGHEST)
    y = jnp.maximum(x_ref[...] * sc + sh, 0.0)
    if has_res:
        y = y + res_ref[...]
    out_ref[...] = y


def _gn_apply(x, s1, v, cnt, gn_w, gn_b, gn_ms, batch2, res=None):
    has_res = res is not None
    in_specs = [
        pl.BlockSpec((R, H), lambda i: (i, 0)),
        pl.BlockSpec((G, H), lambda i: (0, 0)),
        pl.BlockSpec((G, H), lambda i: (0, 0)),
        pl.BlockSpec((G, 1), lambda i: (0, 0)),
        pl.BlockSpec((1, H), lambda i: (0, 0)),
        pl.BlockSpec((1, H), lambda i: (0, 0)),
        pl.BlockSpec((1, H), lambda i: (0, 0)),
        pl.BlockSpec((R, 1), lambda i: (i, 0)),
    ]
    args = [x, s1, v, cnt, gn_w.reshape(1, H), gn_b.reshape(1, H),
            gn_ms.reshape(1, H), batch2]
    if has_res:
        in_specs.append(pl.BlockSpec((R, H), lambda i: (i, 0)))
        args.append(res)
    return pl.pallas_call(
        functools.partial(_gn_body, has_res),
        grid=(GRID,),
        in_specs=in_specs,
        out_specs=[
            pl.BlockSpec((R, H), lambda i: (i, 0)),
            pl.BlockSpec((G, H), lambda i: (0, 0)),
            pl.BlockSpec((G, H), lambda i: (0, 0)),
        ],
        out_shape=[
            jax.ShapeDtypeStruct((N, H), jnp.float32),
            jax.ShapeDtypeStruct((G, H), jnp.float32),
            jax.ShapeDtypeStruct((G, H), jnp.float32),
        ],
    )(*args)[0]


def _final_body(h1_ref, h2_ref, wg1_ref, wg2_ref, bg_ref, wh_ref, bh_ref,
                g2_ref, b2_ref, wf1_ref, bf1_ref, g3_ref, b3_ref, wf2_ref,
                bf2_ref, out_ref):
    h1 = h1_ref[...]
    h2 = h2_ref[...]
    z = (jnp.dot(h1, wg1_ref[...], preferred_element_type=jnp.float32, precision=lax.Precision.HIGHEST)
         + jnp.dot(h2, wg2_ref[...], preferred_element_type=jnp.float32, precision=lax.Precision.HIGHEST)
         + bg_ref[...])
    gate = _rcp(1.0 + jnp.exp(-z))
    h = gate * h1 + (1.0 - gate) * h2
    y = jnp.dot(h, wh_ref[...], preferred_element_type=jnp.float32, precision=lax.Precision.HIGHEST) + bh_ref[...]
    y = jnp.maximum(_ln(y, g2_ref[...], b2_ref[...]), 0.0)
    y = jnp.dot(y, wf1_ref[...], preferred_element_type=jnp.float32, precision=lax.Precision.HIGHEST) + bf1_ref[...]
    y = jnp.maximum(_ln(y, g3_ref[...], b3_ref[...]), 0.0)
    out_ref[...] = jnp.dot(y, wf2_ref[...], preferred_element_type=jnp.float32, precision=lax.Precision.HIGHEST) + bf2_ref[...]


def _final(h1, h2, W_gate, b_gate, W_head, b_head, ln2_g, ln2_b, W_fc1, b_fc1,
           ln3_g, ln3_b, W_fc2, b_fc2):
    H2 = H // 2
    full = lambda shape: pl.BlockSpec(shape, lambda i: (0, 0))
    return pl.pallas_call(
        _final_body,
        grid=(GRID,),
        in_specs=[
            pl.BlockSpec((R, H), lambda i: (i, 0)),
            pl.BlockSpec((R, H), lambda i: (i, 0)),
            full((H, H)), full((H, H)), full((1, H)),
            full((H, H)), full((1, H)),
            full((1, H)), full((1, H)),
            full((H, H2)), full((1, H2)),
            full((1, H2)), full((1, H2)),
            full((H2, 1)), full((1, 1)),
        ],
        out_specs=pl.BlockSpec((R, 1), lambda i: (i, 0)),
        out_shape=jax.ShapeDtypeStruct((N, 1), jnp.float32),
    )(h1, h2, W_gate[:, :H].T, W_gate[:, H:].T, b_gate.reshape(1, H),
      W_head.T, b_head.reshape(1, H), ln2_g.reshape(1, H), ln2_b.reshape(1, H),
      W_fc1.T, b_fc1.reshape(1, H2), ln3_g.reshape(1, H2), ln3_b.reshape(1, H2),
      W_fc2.T, b_fc2.reshape(1, 1))


def kernel(rna_embeddings, ss_emb, W_fuse, b_fuse, ln1_g, ln1_b, W_gat,
           att_src, att_dst, b_gat, gn_w, gn_b, gn_ms, W_gate, b_gate, W_head,
           b_head, ln2_g, ln2_b, W_fc1, b_fc1, ln3_g, ln3_b, W_fc2, b_fc2,
           edge_index, batch):
    loop = jnp.arange(N, dtype=edge_index.dtype)
    srcp = jnp.pad(jnp.concatenate([edge_index[0], loop]), (0, EPAD + C - EL))
    dstp = jnp.pad(jnp.concatenate([edge_index[1], loop]), (0, EPAD + C - EL))
    batch2 = batch.reshape(N, 1)

    h = _fuse(rna_embeddings, ss_emb, W_fuse, b_fuse, ln1_g, ln1_b)

    def gat_block(x, res):
        hw, s, d, ms, md = _gatpre(x, W_gat, att_src, att_dst)
        m = ms[0, 0] + md[0, 0]
        m = jnp.where(m > 0.0, m, 0.2 * m)
        m16 = jnp.full((16,), m, jnp.float32)
        acc, den = _gat_agg(hw, s.reshape(N), d.reshape(N), srcp, dstp, m16)
        gat, s1, cnt = _combstats(acc, den.reshape(NC, NPAD, 1), b_gat, batch2)
        v = _varstats(gat, s1, cnt, gn_ms, batch2)
        return _gn_apply(gat, s1, v, cnt, gn_w, gn_b, gn_ms, batch2, res=res)

    h1 = gat_block(h, None)
    h2 = gat_block(h1, h)
    out = _final(h1, h2, W_gate, b_gate, W_head, b_head, ln2_g, ln2_b,
                 W_fc1, b_fc1, ln3_g, ln3_b, W_fc2, b_fc2)
    return out.reshape(N)


# speedup vs baseline: 40.6549x; 1.0795x over previous
import functools

import jax
import jax.numpy as jnp
from jax import lax
from jax.experimental import pallas as pl
from jax.experimental.pallas import tpu as pltpu
from jax.experimental.pallas import tpu_sc as plsc

N = 10000
H = 128
G = 64
EL = 650000
R = 2000
GRID = N // R

NC = 2
NS = 16
C = 96
NCH = 212
EPW = NCH * C
E_HALF = EPW * NS
EPAD = E_HALF * NC
NPAD = 10240
RPT = NPAD // NS


def _rcp(x):
    r = 1.0 / x
    return r * (2.0 - x * r)


def _rsqrt(x):
    r = lax.rsqrt(x)
    return r * (1.5 - 0.5 * x * r * r)


def _ln(x, g, b):
    m = jnp.mean(x, axis=-1, keepdims=True)
    v = jnp.mean((x - m) ** 2, axis=-1, keepdims=True)
    return (x - m) * _rsqrt(v + 1e-5) * g + b


def _gatpre_block(h, w_ref, as_ref, ad_ref, hw_ref, s_ref, d_ref, ms_ref, md_ref, i):
    hw = jnp.dot(h, w_ref[...], preferred_element_type=jnp.float32, precision=lax.Precision.HIGHEST)
    hw_ref[...] = hw
    s = jnp.dot(hw, as_ref[...], preferred_element_type=jnp.float32, precision=lax.Precision.HIGHEST)
    d = jnp.dot(hw, ad_ref[...], preferred_element_type=jnp.float32, precision=lax.Precision.HIGHEST)
    s_ref[...] = s
    d_ref[...] = d

    @pl.when(i == 0)
    def _():
        ms_ref[...] = jnp.full((1, 1), -1e30, jnp.float32)
        md_ref[...] = jnp.full((1, 1), -1e30, jnp.float32)

    ms_ref[...] = jnp.maximum(ms_ref[...], jnp.max(s).reshape(1, 1))
    md_ref[...] = jnp.maximum(md_ref[...], jnp.max(d).reshape(1, 1))


def _gn_block(x_ref, s1_ref, v_ref, cnt_ref, w_ref, b_ref, ms_ref, batch_ref,
              scale_ref, shift_ref, i):
    @pl.when(i == 0)
    def _():
        cnt = jnp.maximum(cnt_ref[...], 1.0)
        mm = s1_ref[...] / cnt * ms_ref[...]
        var = v_ref[...] / cnt
        rstd = _rsqrt(var + 1e-5)
        scale = w_ref[...] * rstd
        scale_ref[...] = scale
        shift_ref[...] = b_ref[...] - scale * mm

    oh = (batch_ref[...] == lax.broadcasted_iota(jnp.int32, (1, G), 1)
          ).astype(jnp.float32)
    sc = jnp.dot(oh, scale_ref[...], preferred_element_type=jnp.float32, precision=lax.Precision.HIGHEST)
    sh = jnp.dot(oh, shift_ref[...], preferred_element_type=jnp.float32, precision=lax.Precision.HIGHEST)
    return jnp.maximum(x_ref[...] * sc + sh, 0.0)


_W_FULL = lambda shape: pl.BlockSpec(shape, lambda i: (0, 0))


def _pre1_body(rna_ref, ss_ref, w1_ref, w2_ref, b_ref, g_ref, bb_ref,
               w_ref, as_ref, ad_ref,
               h_ref, hw_ref, s_ref, d_ref, ms_ref, md_ref):
    x = jnp.dot(rna_ref[...], w1_ref[...], preferred_element_type=jnp.float32, precision=lax.Precision.HIGHEST)
    x = x + jnp.dot(ss_ref[...], w2_ref[...], preferred_element_type=jnp.float32, precision=lax.Precision.HIGHEST)
    x = x + b_ref[...]
    h = jnp.maximum(_ln(x, g_ref[...], bb_ref[...]), 0.0)
    h_ref[...] = h
    _gatpre_block(h, w_ref, as_ref, ad_ref, hw_ref, s_ref, d_ref, ms_ref,
                  md_ref, pl.program_id(0))


def _pre1(rna, ss, W_fuse, b_fuse, g, b, W_gat, att_src, att_dst):
    w1 = W_fuse[:, :rna.shape[1]].T
    w2 = W_fuse[:, rna.shape[1]:].T
    return pl.pallas_call(
        _pre1_body,
        grid=(GRID,),
        in_specs=[
            pl.BlockSpec((R, rna.shape[1]), lambda i: (i, 0)),
            pl.BlockSpec((R, ss.shape[1]), lambda i: (i, 0)),
            _W_FULL(w1.shape), _W_FULL(w2.shape),
            _W_FULL((1, H)), _W_FULL((1, H)), _W_FULL((1, H)),
            _W_FULL((H, H)), _W_FULL((H, 1)), _W_FULL((H, 1)),
        ],
        out_specs=[
            pl.BlockSpec((R, H), lambda i: (i, 0)),
            pl.BlockSpec((R, H), lambda i: (i, 0)),
            pl.BlockSpec((R, 1), lambda i: (i, 0)),
            pl.BlockSpec((R, 1), lambda i: (i, 0)),
            _W_FULL((1, 1)), _W_FULL((1, 1)),
        ],
        out_shape=[
            jax.ShapeDtypeStruct((N, H), jnp.float32),
            jax.ShapeDtypeStruct((N, H), jnp.float32),
            jax.ShapeDtypeStruct((N, 1), jnp.float32),
            jax.ShapeDtypeStruct((N, 1), jnp.float32),
            jax.ShapeDtypeStruct((1, 1), jnp.float32),
            jax.ShapeDtypeStruct((1, 1), jnp.float32),
        ],
    )(rna, ss, w1, w2, b_fuse.reshape(1, H), g.reshape(1, H), b.reshape(1, H),
      W_gat.T, att_src.reshape(H, 1), att_dst.reshape(H, 1))


def _mid_body(x_ref, s1_ref, v_ref, cnt_ref, gw_ref, gb_ref, gms_ref,
              batch_ref, w_ref, as_ref, ad_ref,
              h1_ref, hw_ref, s_ref, d_ref, ms_ref, md_ref,
              scale_ref, shift_ref):
    i = pl.program_id(0)
    y = _gn_block(x_ref, s1_ref, v_ref, cnt_ref, gw_ref, gb_ref, gms_ref,
                  batch_ref, scale_ref, shift_ref, i)
    h1_ref[...] = y
    _gatpre_block(y, w_ref, as_ref, ad_ref, hw_ref, s_ref, d_ref, ms_ref,
                  md_ref, i)


def _mid(x, s1, v, cnt, gn_w, gn_b, gn_ms, batch2, W_gat, att_src, att_dst):
    return pl.pallas_call(
        _mid_body,
        grid=(GRID,),
        in_specs=[
            pl.BlockSpec((R, H), lambda i: (i, 0)),
            _W_FULL((G, H)), _W_FULL((G, H)), _W_FULL((G, 1)),
            _W_FULL((1, H)), _W_FULL((1, H)), _W_FULL((1, H)),
            pl.BlockSpec((R, 1), lambda i: (i, 0)),
            _W_FULL((H, H)), _W_FULL((H, 1)), _W_FULL((H, 1)),
        ],
        out_specs=[
            pl.BlockSpec((R, H), lambda i: (i, 0)),
            pl.BlockSpec((R, H), lambda i: (i, 0)),
            pl.BlockSpec((R, 1), lambda i: (i, 0)),
            pl.BlockSpec((R, 1), lambda i: (i, 0)),
            _W_FULL((1, 1)), _W_FULL((1, 1)),
        ],
        out_shape=[
            jax.ShapeDtypeStruct((N, H), jnp.float32),
            jax.ShapeDtypeStruct((N, H), jnp.float32),
            jax.ShapeDtypeStruct((N, 1), jnp.float32),
            jax.ShapeDtypeStruct((N, 1), jnp.float32),
            jax.ShapeDtypeStruct((1, 1), jnp.float32),
            jax.ShapeDtypeStruct((1, 1), jnp.float32),
        ],
        scratch_shapes=[
            pltpu.VMEM((G, H), jnp.float32),
            pltpu.VMEM((G, H), jnp.float32),
        ],
    )(x, s1, v, cnt, gn_w.reshape(1, H), gn_b.reshape(1, H),
      gn_ms.reshape(1, H), batch2, W_gat.T, att_src.reshape(H, 1),
      att_dst.reshape(H, 1))


def _gat_agg_body(hw_hbm, asrc_hbm, adst_hbm, src_hbm, dst_hbm, m_hbm,
                  acc_out, den_out,
                  asrc_v, adst_v, srcc_v, dstc_v, dsts_v, ea_v, eas_v, rows_v,
                  m_v, acc_sh, den_sh,
                  sis0, sis1, sid0, sid1, sg0, sg1, ssc0, ssc1, sd0, sd1):
    cid = lax.axis_index("c")
    sid = lax.axis_index("s")
    si_s = (sis0, sis1)
    si_d = (sid0, sid1)
    sg = (sg0, sg1)
    ssc = (ssc0, ssc1)
    sd = (sd0, sd1)
    zero16 = jnp.zeros((16,), jnp.float32)

    def _zb(r, carry):
        for kk in range(H // 16):
            rows_v[0, r, pl.ds(kk * 16, 16)] = zero16
        return carry
    lax.fori_loop(0, 64, _zb, 0)
    for kk in range(C // 16):
        ea_v[0, pl.ds(kk * 16, 16)] = zero16
    for j in range(RPT // 64):
        pltpu.sync_copy(rows_v.at[0, pl.ds(0, 64)],
                        acc_sh.at[pl.ds(sid * RPT + j * 64, 64)])
        pltpu.sync_copy(ea_v.at[0, pl.ds(0, 64)],
                        den_sh.at[pl.ds(sid * RPT + j * 64, 64)])

    pltpu.sync_copy(asrc_hbm, asrc_v)
    pltpu.sync_copy(adst_hbm, adst_v)
    pltpu.sync_copy(m_hbm, m_v)
    ebase = cid * E_HALF + sid * EPW
    plsc.subcore_barrier()
    mvec = m_v[...]

    pltpu.async_copy(src_hbm.at[pl.ds(ebase, C)], srcc_v.at[0], si_s[0])
    pltpu.async_copy(dst_hbm.at[pl.ds(ebase, C)], dstc_v.at[0], si_d[0])

    def _do_chunk(i2, b, eb):
        pltpu.make_async_copy(src_hbm.at[pl.ds(eb, C)], srcc_v.at[b], si_s[b]).wait()
        pltpu.make_async_copy(dst_hbm.at[pl.ds(eb, C)], dstc_v.at[b], si_d[b]).wait()
        b1 = 1 - b
        pltpu.async_copy(src_hbm.at[pl.ds(eb + C, C)], srcc_v.at[b1], si_s[b1])
        pltpu.async_copy(dst_hbm.at[pl.ds(eb + C, C)], dstc_v.at[b1], si_d[b1])

        @pl.when(i2 > 0)
        def _():
            pltpu.make_async_copy(rows_v.at[b], acc_sh.at[dsts_v.at[b]], ssc[b]).wait()
            pltpu.make_async_copy(eas_v.at[b], den_sh.at[dsts_v.at[b]], sd[b]).wait()

        gath = pltpu.async_copy(hw_hbm.at[srcc_v.at[b]], rows_v.at[b], sg[b])
        for kk in range(C // 16):
            si = srcc_v[b, pl.ds(kk * 16, 16)]
            di = dstc_v[b, pl.ds(kk * 16, 16)]
            a = plsc.load_gather(asrc_v, [si]) + plsc.load_gather(adst_v, [di])
            a = jnp.where(a > 0.0, a, 0.2 * a)
            e = jnp.exp(a - mvec)
            eid = eb + kk * 16 + lax.iota(jnp.int32, 16)
            e = jnp.where(eid < EL, e, 0.0)
            ea_v[b, pl.ds(kk * 16, 16)] = e
        gath.wait()

        def _scale(jj, c2):
            e16 = ea_v[b, pl.ds(jj * 16, 16)]
            for t in range(16):
                r = jj * 16 + t
                c0 = e16[t]
                for kk in range(H // 16):
                    rows_v[b, r, pl.ds(kk * 16, 16)] = rows_v[b, r, pl.ds(kk * 16, 16)] * c0
            return c2
        lax.fori_loop(0, C // 16, _scale, 0)
        for kk in range(C // 16):
            dsts_v[b, pl.ds(kk * 16, 16)] = dstc_v[b, pl.ds(kk * 16, 16)]
            eas_v[b, pl.ds(kk * 16, 16)] = ea_v[b, pl.ds(kk * 16, 16)]
        pltpu.async_copy(rows_v.at[b], acc_sh.at[dsts_v.at[b]], ssc[b], add=True)
        pltpu.async_copy(eas_v.at[b], den_sh.at[dsts_v.at[b]], sd[b], add=True)

    def _pair(i2, carry):
        eb = ebase + i2 * (2 * C)
        _do_chunk(i2, 0, eb)
        _do_chunk(i2, 1, eb + C)
        return carry

    lax.fori_loop(0, NCH // 2, _pair, 0)

    for b in range(2):
        pltpu.make_async_copy(rows_v.at[b], acc_sh.at[dsts_v.at[b]], ssc[b]).wait()
        pltpu.make_async_copy(eas_v.at[b], den_sh.at[dsts_v.at[b]], sd[b]).wait()
    pltpu.make_async_copy(src_hbm.at[pl.ds(ebase, C)], srcc_v.at[0], si_s[0]).wait()
    pltpu.make_async_copy(dst_hbm.at[pl.ds(ebase, C)], dstc_v.at[0], si_d[0]).wait()

    plsc.subcore_barrier()
    for j in range(RPT // 64):
        pltpu.sync_copy(acc_sh.at[pl.ds(sid * RPT + j * 64, 64)],
                        acc_out.at[cid, pl.ds(sid * RPT + j * 64, 64)])
    pltpu.sync_copy(den_sh.at[pl.ds(sid * RPT, RPT)],
                    den_out.at[cid, pl.ds(sid * RPT, RPT)])


def _gat_agg(hw, asrc, adst, srcp, dstp, m16):
    mesh = plsc.VectorSubcoreMesh(core_axis_name="c", subcore_axis_name="s",
                                  num_cores=NC, num_subcores=NS)
    kfn = pl.kernel(
        _gat_agg_body,
        out_type=(jax.ShapeDtypeStruct((NC, NPAD, H), jnp.float32),
                  jax.ShapeDtypeStruct((NC, NPAD), jnp.float32)),
        mesh=mesh,
        compiler_params=pltpu.CompilerParams(needs_layout_passes=False),
        scratch_types=[
            pltpu.VMEM((N,), jnp.float32),
            pltpu.VMEM((N,), jnp.float32),
            pltpu.VMEM((2, C), jnp.int32),
            pltpu.VMEM((2, C), jnp.int32),
            pltpu.VMEM((2, C), jnp.int32),
            pltpu.VMEM((2, C), jnp.float32),
            pltpu.VMEM((2, C), jnp.float32),
            pltpu.VMEM((2, C, H), jnp.float32),
            pltpu.VMEM((16,), jnp.float32),
            pltpu.VMEM_SHARED((NPAD, H), jnp.float32),
            pltpu.VMEM_SHARED((NPAD,), jnp.float32),
        ] + [pltpu.SemaphoreType.DMA] * 10,
    )
    return kfn(hw, asrc, adst, srcp, dstp, m16)


def _comb_body(acc_ref, den_ref, b_ref, batch_ref, gat_ref, s1_ref, cnt_ref):
    a = acc_ref[0] + acc_ref[1]
    den = den_ref[0] + den_ref[1]
    gat = a * _rcp(den) + b_ref[...]
    gat_ref[...] = gat
    oh = (batch_ref[...] == lax.broadcasted_iota(jnp.int32, (1, G), 1)
          ).astype(jnp.float32)
    i = pl.program_id(0)

    @pl.when(i == 0)
    def _():
        s1_ref[...] = jnp.zeros_like(s1_ref)
        cnt_ref[...] = jnp.zeros_like(cnt_ref)

    dn = (((0,), (0,)), ((), ()))
    s1_ref[...] += lax.dot_general(oh, gat, dn, preferred_element_type=jnp.float32, precision=lax.Precision.HIGHEST)
    cnt_ref[...] += jnp.sum(oh, axis=0, keepdims=True).T


def _combstats(acc, den3, b_gat, batch2):
    return pl.pallas_call(
        _comb_body,
        grid=(GRID,),
        in_specs=[
            pl.BlockSpec((NC, R, H), lambda i: (0, i, 0)),
            pl.BlockSpec((NC, R, 1), lambda i: (0, i, 0)),
            pl.BlockSpec((1, H), lambda i: (0, 0)),
            pl.BlockSpec((R, 1), lambda i: (i, 0)),
        ],
        out_specs=[
            pl.BlockSpec((R, H), lambda i: (i, 0)),
            pl.BlockSpec((G, H), lambda i: (0, 0)),
            pl.BlockSpec((G, 1), lambda i: (0, 0)),
        ],
        out_shape=[
            jax.ShapeDtypeStruct((N, H), jnp.float32),
            jax.ShapeDtypeStruct((G, H), jnp.float32),
            jax.ShapeDtypeStruct((G, 1), jnp.float32),
        ],
    )(acc, den3, b_gat.reshape(1, H), batch2)


def _var_body(x_ref, s1_ref, cnt_ref, ms_ref, batch_ref, v_ref, mean_ref):
    i = pl.program_id(0)

    @pl.when(i == 0)
    def _():
        cnt = jnp.maximum(cnt_ref[...], 1.0)
        mean_ref[...] = s1_ref[...] / cnt * ms_ref[...]
        v_ref[...] = jnp.zeros_like(v_ref)

    oh = (batch_ref[...] == lax.broadcasted_iota(jnp.int32, (1, G), 1)
          ).astype(jnp.float32)
    mm = jnp.dot(oh, mean_ref[...], preferred_element_type=jnp.float32, precision=lax.Precision.HIGHEST)
    d = x_ref[...] - mm
    dn = (((0,), (0,)), ((), ()))
    v_ref[...] += lax.dot_general(oh, d * d, dn, preferred_element_type=jnp.float32, precision=lax.Precision.HIGHEST)


def _varstats(x, s1, cnt, gn_ms, batch2):
    return pl.pallas_call(
        _var_body,
        grid=(GRID,),
        in_specs=[
            pl.BlockSpec((R, H), lambda i: (i, 0)),
            pl.BlockSpec((G, H), lambda i: (0, 0)),
            pl.BlockSpec((G, 1), lambda i: (0, 0)),
            pl.BlockSpec((1, H), lambda i: (0, 0)),
            pl.BlockSpec((R, 1), lambda i: (i, 0)),
        ],
        out_specs=[
            pl.BlockSpec((G, H), lambda i: (0, 0)),
            pl.BlockSpec((G, H), lambda i: (0, 0)),
        ],
        out_shape=[
            jax.ShapeDtypeStruct((G, H), jnp.float32),
            jax.ShapeDtypeStruct((G, H), jnp.float32),
        ],
    )(x, s1, cnt, gn_ms.reshape(1, H), batch2)[0]


def _tail_body(x_ref, s1_ref, v_ref, cnt_ref, gw_ref, gb_ref, gms_ref,
               batch_ref, res_ref, h1_ref,
               wg1_ref, wg2_ref, bg_ref, wh_ref, bh_ref, g2_ref, b2_ref,
               wf1_ref, bf1_ref, g3_ref, b3_ref, wf2_ref, bf2_ref,
               out_ref, scale_ref, shift_ref):
    i = pl.program_id(0)
    y = _gn_block(x_ref, s1_ref, v_ref, cnt_ref, gw_ref, gb_ref, gms_ref,
                  batch_ref, scale_ref, shift_ref, i)
    h2 = y + res_ref[...]
    h1 = h1_ref[...]
    z = (jnp.dot(h1, wg1_ref[...], preferred_element_type=jnp.float32, precision=lax.Precision.HIGHEST)
         + jnp.dot(h2, wg2_ref[...], preferred_element_type=jnp.float32, precision=lax.Precision.HIGHEST)
         + bg_ref[...])
    gate = _rcp(1.0 + jnp.exp(-z))
    h = gate * h1 + (1.0 - gate) * h2
    y = jnp.dot(h, wh_ref[...], preferred_element_type=jnp.float32, precision=lax.Precision.HIGHEST) + bh_ref[...]
    y = jnp.maximum(_ln(y, g2_ref[...], b2_ref[...]), 0.0)
    y = jnp.dot(y, wf1_ref[...], preferred_element_type=jnp.float32, precision=lax.Precision.HIGHEST) + bf1_ref[...]
    y = jnp.maximum(_ln(y, g3_ref[...], b3_ref[...]), 0.0)
    out_ref[...] = jnp.dot(y, wf2_ref[...], preferred_element_type=jnp.float32, precision=lax.Precision.HIGHEST) + bf2_ref[...]


def _tail(x, s1, v, cnt, gn_w, gn_b, gn_ms, batch2, res, h1, W_gate, b_gate,
          W_head, b_head, ln2_g, ln2_b, W_fc1, b_fc1, ln3_g, ln3_b, W_fc2,
          b_fc2):
    H2 = H // 2
    return pl.pallas_call(
        _tail_body,
        grid=(GRID,),
        in_specs=[
            pl.BlockSpec((R, H), lambda i: (i, 0)),
            _W_FULL((G, H)), _W_FULL((G, H)), _W_FULL((G, 1)),
            _W_FULL((1, H)), _W_FULL((1, H)), _W_FULL((1, H)),
            pl.BlockSpec((R, 1), lambda i: (i, 0)),
            pl.BlockSpec((R, H), lambda i: (i, 0)),
            pl.BlockSpec((R, H), lambda i: (i, 0)),
            _W_FULL((H, H)), _W_FULL((H, H)), _W_FULL((1, H)),
            _W_FULL((H, H)), _W_FULL((1, H)),
            _W_FULL((1, H)), _W_FULL((1, H)),
            _W_FULL((H, H2)), _W_FULL((1, H2)),
            _W_FULL((1, H2)), _W_FULL((1, H2)),
            _W_FULL((H2, 1)), _W_FULL((1, 1)),
        ],
        out_specs=pl.BlockSpec((R, 1), lambda i: (i, 0)),
        out_shape=jax.ShapeDtypeStruct((N, 1), jnp.float32),
        scratch_shapes=[
            pltpu.VMEM((G, H), jnp.float32),
            pltpu.VMEM((G, H), jnp.float32),
        ],
    )(x, s1, v, cnt, gn_w.reshape(1, H), gn_b.reshape(1, H),
      gn_ms.reshape(1, H), batch2, res, h1,
      W_gate[:, :H].T, W_gate[:, H:].T, b_gate.reshape(1, H),
      W_head.T, b_head.reshape(1, H), ln2_g.reshape(1, H), ln2_b.reshape(1, H),
      W_fc1.T, b_fc1.reshape(1, H2), ln3_g.reshape(1, H2), ln3_b.reshape(1, H2),
      W_fc2.T, b_fc2.reshape(1, 1))


def kernel(rna_embeddings, ss_emb, W_fuse, b_fuse, ln1_g, ln1_b, W_gat,
           att_src, att_dst, b_gat, gn_w, gn_b, gn_ms, W_gate, b_gate, W_head,
           b_head, ln2_g, ln2_b, W_fc1, b_fc1, ln3_g, ln3_b, W_fc2, b_fc2,
           edge_index, batch):
    loop = jnp.arange(N, dtype=edge_index.dtype)
    srcp = jnp.pad(jnp.concatenate([edge_index[0], loop]), (0, EPAD + C - EL))
    dstp = jnp.pad(jnp.concatenate([edge_index[1], loop]), (0, EPAD + C - EL))
    batch2 = batch.reshape(N, 1)

    def edge_pass(hw, s, d, ms, md):
        m = ms[0, 0] + md[0, 0]
        m = jnp.where(m > 0.0, m, 0.2 * m)
        m16 = jnp.full((16,), m, jnp.float32)
        acc, den = _gat_agg(hw, s.reshape(N), d.reshape(N), srcp, dstp, m16)
        gat, s1, cnt = _combstats(acc, den.reshape(NC, NPAD, 1), b_gat, batch2)
        v = _varstats(gat, s1, cnt, gn_ms, batch2)
        return gat, s1, v, cnt

    h, hw1, s1a, d1a, ms1, md1 = _pre1(rna_embeddings, ss_emb, W_fuse, b_fuse,
                                       ln1_g, ln1_b, W_gat, att_src, att_dst)
    gat1, s1_1, v1, cnt1 = edge_pass(hw1, s1a, d1a, ms1, md1)
    h1, hw2, s2a, d2a, ms2, md2 = _mid(gat1, s1_1, v1, cnt1, gn_w, gn_b, gn_ms,
                                       batch2, W_gat, att_src, att_dst)
    gat2, s1_2, v2, cnt2 = edge_pass(hw2, s2a, d2a, ms2, md2)
    out = _tail(gat2, s1_2, v2, cnt2, gn_w, gn_b, gn_ms, batch2, h, h1,
                W_gate, b_gate, W_head, b_head, ln2_g, ln2_b, W_fc1, b_fc1,
                ln3_g, ln3_b, W_fc2, b_fc2)
    return out.reshape(N)


# speedup vs baseline: 43.3749x; 1.0669x over previous
import jax
import jax.numpy as jnp
from jax import lax
from jax.experimental import pallas as pl
from jax.experimental.pallas import tpu as pltpu
from jax.experimental.pallas import tpu_sc as plsc

N = 10000
H = 128
G = 64
EL = 650000
R = 2000
GRID = N // R

NC = 2
NS = 16
C = 96
NCH = 212
EPW = NCH * C
E_HALF = EPW * NS
EPAD = E_HALF * NC
NPAD = 10240
RPT = NPAD // NS


def _bdot(a, b):
    return jnp.dot(a.astype(jnp.bfloat16), b.astype(jnp.bfloat16),
                   preferred_element_type=jnp.float32)


def _rcp(x):
    r = 1.0 / x
    return r * (2.0 - x * r)


def _rsqrt(x):
    r = lax.rsqrt(x)
    return r * (1.5 - 0.5 * x * r * r)


def _ln(x, g, b):
    m = jnp.mean(x, axis=-1, keepdims=True)
    v = jnp.mean((x - m) ** 2, axis=-1, keepdims=True)
    return (x - m) * _rsqrt(v + 1e-5) * g + b


def _gatpre_block(h, w_ref, as_ref, ad_ref, hw_ref, s_ref, d_ref, ms_ref, md_ref, i):
    hw = _bdot(h, w_ref[...])
    hw_ref[...] = hw
    s = _bdot(hw, as_ref[...])
    d = _bdot(hw, ad_ref[...])
    s_ref[...] = s
    d_ref[...] = d

    @pl.when(i == 0)
    def _():
        ms_ref[...] = jnp.full((1, 1), -1e30, jnp.float32)
        md_ref[...] = jnp.full((1, 1), -1e30, jnp.float32)

    ms_ref[...] = jnp.maximum(ms_ref[...], jnp.max(s).reshape(1, 1))
    md_ref[...] = jnp.maximum(md_ref[...], jnp.max(d).reshape(1, 1))


def _gn_block(x_ref, s1_ref, v_ref, cnt_ref, w_ref, b_ref, ms_ref, batch_ref,
              scale_ref, shift_ref, i):
    @pl.when(i == 0)
    def _():
        cnt = jnp.maximum(cnt_ref[...], 1.0)
        mm = s1_ref[...] / cnt * ms_ref[...]
        var = v_ref[...] / cnt
        rstd = _rsqrt(var + 1e-5)
        scale = w_ref[...] * rstd
        scale_ref[...] = scale
        shift_ref[...] = b_ref[...] - scale * mm

    oh = (batch_ref[...] == lax.broadcasted_iota(jnp.int32, (1, G), 1)
          ).astype(jnp.float32)
    sc = jnp.dot(oh, scale_ref[...], preferred_element_type=jnp.float32, precision=lax.Precision.HIGHEST)
    sh = jnp.dot(oh, shift_ref[...], preferred_element_type=jnp.float32, precision=lax.Precision.HIGHEST)
    return jnp.maximum(x_ref[...] * sc + sh, 0.0)


_W_FULL = lambda shape: pl.BlockSpec(shape, lambda i: (0, 0))


def _pre1_body(rna_ref, ss_ref, w1_ref, w2_ref, b_ref, g_ref, bb_ref,
               w_ref, as_ref, ad_ref,
               h_ref, hw_ref, s_ref, d_ref, ms_ref, md_ref):
    x = _bdot(rna_ref[...], w1_ref[...])
    x = x + _bdot(ss_ref[...], w2_ref[...])
    x = x + b_ref[...]
    h = jnp.maximum(_ln(x, g_ref[...], bb_ref[...]), 0.0)
    h_ref[...] = h
    _gatpre_block(h, w_ref, as_ref, ad_ref, hw_ref, s_ref, d_ref, ms_ref,
                  md_ref, pl.program_id(0))


def _pre1(rna, ss, W_fuse, b_fuse, g, b, W_gat, att_src, att_dst):
    w1 = W_fuse[:, :rna.shape[1]].T
    w2 = W_fuse[:, rna.shape[1]:].T
    return pl.pallas_call(
        _pre1_body,
        grid=(GRID,),
        in_specs=[
            pl.BlockSpec((R, rna.shape[1]), lambda i: (i, 0)),
            pl.BlockSpec((R, ss.shape[1]), lambda i: (i, 0)),
            _W_FULL(w1.shape), _W_FULL(w2.shape),
            _W_FULL((1, H)), _W_FULL((1, H)), _W_FULL((1, H)),
            _W_FULL((H, H)), _W_FULL((H, 1)), _W_FULL((H, 1)),
        ],
        out_specs=[
            pl.BlockSpec((R, H), lambda i: (i, 0)),
            pl.BlockSpec((R, H), lambda i: (i, 0)),
            pl.BlockSpec((R, 1), lambda i: (i, 0)),
            pl.BlockSpec((R, 1), lambda i: (i, 0)),
            _W_FULL((1, 1)), _W_FULL((1, 1)),
        ],
        out_shape=[
            jax.ShapeDtypeStruct((N, H), jnp.float32),
            jax.ShapeDtypeStruct((N, H), jnp.float32),
            jax.ShapeDtypeStruct((N, 1), jnp.float32),
            jax.ShapeDtypeStruct((N, 1), jnp.float32),
            jax.ShapeDtypeStruct((1, 1), jnp.float32),
            jax.ShapeDtypeStruct((1, 1), jnp.float32),
        ],
    )(rna, ss, w1, w2, b_fuse.reshape(1, H), g.reshape(1, H), b.reshape(1, H),
      W_gat.T, att_src.reshape(H, 1), att_dst.reshape(H, 1))


def _mid_body(x_ref, s1_ref, v_ref, cnt_ref, gw_ref, gb_ref, gms_ref,
              batch_ref, w_ref, as_ref, ad_ref,
              h1_ref, hw_ref, s_ref, d_ref, ms_ref, md_ref,
              scale_ref, shift_ref):
    i = pl.program_id(0)
    y = _gn_block(x_ref, s1_ref, v_ref, cnt_ref, gw_ref, gb_ref, gms_ref,
                  batch_ref, scale_ref, shift_ref, i)
    h1_ref[...] = y
    _gatpre_block(y, w_ref, as_ref, ad_ref, hw_ref, s_ref, d_ref, ms_ref,
                  md_ref, i)


def _mid(x, s1, v, cnt, gn_w, gn_b, gn_ms, batch2, W_gat, att_src, att_dst):
    return pl.pallas_call(
        _mid_body,
        grid=(GRID,),
        in_specs=[
            pl.BlockSpec((R, H), lambda i: (i, 0)),
            _W_FULL((G, H)), _W_FULL((G, H)), _W_FULL((G, 1)),
            _W_FULL((1, H)), _W_FULL((1, H)), _W_FULL((1, H)),
            pl.BlockSpec((R, 1), lambda i: (i, 0)),
            _W_FULL((H, H)), _W_FULL((H, 1)), _W_FULL((H, 1)),
        ],
        out_specs=[
            pl.BlockSpec((R, H), lambda i: (i, 0)),
            pl.BlockSpec((R, H), lambda i: (i, 0)),
            pl.BlockSpec((R, 1), lambda i: (i, 0)),
            pl.BlockSpec((R, 1), lambda i: (i, 0)),
            _W_FULL((1, 1)), _W_FULL((1, 1)),
        ],
        out_shape=[
            jax.ShapeDtypeStruct((N, H), jnp.float32),
            jax.ShapeDtypeStruct((N, H), jnp.float32),
            jax.ShapeDtypeStruct((N, 1), jnp.float32),
            jax.ShapeDtypeStruct((N, 1), jnp.float32),
            jax.ShapeDtypeStruct((1, 1), jnp.float32),
            jax.ShapeDtypeStruct((1, 1), jnp.float32),
        ],
        scratch_shapes=[
            pltpu.VMEM((G, H), jnp.float32),
            pltpu.VMEM((G, H), jnp.float32),
        ],
    )(x, s1, v, cnt, gn_w.reshape(1, H), gn_b.reshape(1, H),
      gn_ms.reshape(1, H), batch2, W_gat.T, att_src.reshape(H, 1),
      att_dst.reshape(H, 1))


def _gat_agg_body(hw_hbm, asrc_hbm, adst_hbm, src_hbm, dst_hbm, m_hbm,
                  acc_out, den_out,
                  asrc_v, adst_v, srcc_v, dstc_v, dsts_v, ea_v, eas_v, rows_v,
                  m_v, acc_sh, den_sh,
                  sis0, sis1, sid0, sid1, sg0, sg1, ssc0, ssc1, sd0, sd1):
    cid = lax.axis_index("c")
    sid = lax.axis_index("s")
    si_s = (sis0, sis1)
    si_d = (sid0, sid1)
    sg = (sg0, sg1)
    ssc = (ssc0, ssc1)
    sd = (sd0, sd1)
    zero16 = jnp.zeros((16,), jnp.float32)

    def _zb(r, carry):
        for kk in range(H // 16):
            rows_v[0, r, pl.ds(kk * 16, 16)] = zero16
        return carry
    lax.fori_loop(0, 64, _zb, 0)
    for kk in range(C // 16):
        ea_v[0, pl.ds(kk * 16, 16)] = zero16
    for j in range(RPT // 64):
        pltpu.sync_copy(rows_v.at[0, pl.ds(0, 64)],
                        acc_sh.at[pl.ds(sid * RPT + j * 64, 64)])
        pltpu.sync_copy(ea_v.at[0, pl.ds(0, 64)],
                        den_sh.at[pl.ds(sid * RPT + j * 64, 64)])

    pltpu.sync_copy(asrc_hbm, asrc_v)
    pltpu.sync_copy(adst_hbm, adst_v)
    pltpu.sync_copy(m_hbm, m_v)
    ebase = cid * E_HALF + sid * EPW
    plsc.subcore_barrier()
    mvec = m_v[...]

    pltpu.async_copy(src_hbm.at[pl.ds(ebase, C)], srcc_v.at[0], si_s[0])
    pltpu.async_copy(dst_hbm.at[pl.ds(ebase, C)], dstc_v.at[0], si_d[0])

    def _do_chunk(i2, b, eb):
        pltpu.make_async_copy(src_hbm.at[pl.ds(eb, C)], srcc_v.at[b], si_s[b]).wait()
        pltpu.make_async_copy(dst_hbm.at[pl.ds(eb, C)], dstc_v.at[b], si_d[b]).wait()
        b1 = 1 - b
        pltpu.async_copy(src_hbm.at[pl.ds(eb + C, C)], srcc_v.at[b1], si_s[b1])
        pltpu.async_copy(dst_hbm.at[pl.ds(eb + C, C)], dstc_v.at[b1], si_d[b1])

        @pl.when(i2 > 0)
        def _():
            pltpu.make_async_copy(rows_v.at[b], acc_sh.at[dsts_v.at[b]], ssc[b]).wait()
            pltpu.make_async_copy(eas_v.at[b], den_sh.at[dsts_v.at[b]], sd[b]).wait()

        gath = pltpu.async_copy(hw_hbm.at[srcc_v.at[b]], rows_v.at[b], sg[b])
        for kk in range(C // 16):
            si = srcc_v[b, pl.ds(kk * 16, 16)]
            di = dstc_v[b, pl.ds(kk * 16, 16)]
            a = plsc.load_gather(asrc_v, [si]) + plsc.load_gather(adst_v, [di])
            a = jnp.where(a > 0.0, a, 0.2 * a)
            e = jnp.exp(a - mvec)
            eid = eb + kk * 16 + lax.iota(jnp.int32, 16)
            e = jnp.where(eid < EL, e, 0.0)
            ea_v[b, pl.ds(kk * 16, 16)] = e
        gath.wait()

        def _scale(jj, c2):
            e16 = ea_v[b, pl.ds(jj * 16, 16)]
            for t in range(16):
                r = jj * 16 + t
                c0 = e16[t]
                for kk in range(H // 16):
                    rows_v[b, r, pl.ds(kk * 16, 16)] = rows_v[b, r, pl.ds(kk * 16, 16)] * c0
            return c2
        lax.fori_loop(0, C // 16, _scale, 0)
        for kk in range(C // 16):
            dsts_v[b, pl.ds(kk * 16, 16)] = dstc_v[b, pl.ds(kk * 16, 16)]
            eas_v[b, pl.ds(kk * 16, 16)] = ea_v[b, pl.ds(kk * 16, 16)]
        pltpu.async_copy(rows_v.at[b], acc_sh.at[dsts_v.at[b]], ssc[b], add=True)
        pltpu.async_copy(eas_v.at[b], den_sh.at[dsts_v.at[b]], sd[b], add=True)

    def _pair(i2, carry):
        eb = ebase + i2 * (2 * C)
        _do_chunk(i2, 0, eb)
        _do_chunk(i2, 1, eb + C)
        return carry

    lax.fori_loop(0, NCH // 2, _pair, 0)

    for b in range(2):
        pltpu.make_async_copy(rows_v.at[b], acc_sh.at[dsts_v.at[b]], ssc[b]).wait()
        pltpu.make_async_copy(eas_v.at[b], den_sh.at[dsts_v.at[b]], sd[b]).wait()
    pltpu.make_async_copy(src_hbm.at[pl.ds(ebase, C)], srcc_v.at[0], si_s[0]).wait()
    pltpu.make_async_copy(dst_hbm.at[pl.ds(ebase, C)], dstc_v.at[0], si_d[0]).wait()

    plsc.subcore_barrier()
    for j in range(RPT // 64):
        pltpu.sync_copy(acc_sh.at[pl.ds(sid * RPT + j * 64, 64)],
                        acc_out.at[cid, pl.ds(sid * RPT + j * 64, 64)])
    pltpu.sync_copy(den_sh.at[pl.ds(sid * RPT, RPT)],
                    den_out.at[cid, pl.ds(sid * RPT, RPT)])


def _gat_agg(hw, asrc, adst, srcp, dstp, m16):
    mesh = plsc.VectorSubcoreMesh(core_axis_name="c", subcore_axis_name="s",
                                  num_cores=NC, num_subcores=NS)
    kfn = pl.kernel(
        _gat_agg_body,
        out_type=(jax.ShapeDtypeStruct((NC, NPAD, H), jnp.float32),
                  jax.ShapeDtypeStruct((NC, NPAD), jnp.float32)),
        mesh=mesh,
        compiler_params=pltpu.CompilerParams(needs_layout_passes=False),
        scratch_types=[
            pltpu.VMEM((N,), jnp.float32),
            pltpu.VMEM((N,), jnp.float32),
            pltpu.VMEM((2, C), jnp.int32),
            pltpu.VMEM((2, C), jnp.int32),
            pltpu.VMEM((2, C), jnp.int32),
            pltpu.VMEM((2, C), jnp.float32),
            pltpu.VMEM((2, C), jnp.float32),
            pltpu.VMEM((2, C, H), jnp.float32),
            pltpu.VMEM((16,), jnp.float32),
            pltpu.VMEM_SHARED((NPAD, H), jnp.float32),
            pltpu.VMEM_SHARED((NPAD,), jnp.float32),
        ] + [pltpu.SemaphoreType.DMA] * 10,
    )
    return kfn(hw, asrc, adst, srcp, dstp, m16)


def _comb_body(acc_ref, den_ref, b_ref, batch_ref, gat_ref, s1_ref, cnt_ref):
    a = acc_ref[0] + acc_ref[1]
    den = den_ref[0] + den_ref[1]
    gat = a * _rcp(den) + b_ref[...]
    gat_ref[...] = gat
    oh = (batch_ref[...] == lax.broadcasted_iota(jnp.int32, (1, G), 1)
          ).astype(jnp.float32)
    i = pl.program_id(0)

    @pl.when(i == 0)
    def _():
        s1_ref[...] = jnp.zeros_like(s1_ref)
        cnt_ref[...] = jnp.zeros_like(cnt_ref)

    dn = (((0,), (0,)), ((), ()))
    s1_ref[...] += lax.dot_general(oh, gat, dn, preferred_element_type=jnp.float32, precision=lax.Precision.HIGHEST)
    cnt_ref[...] += jnp.sum(oh, axis=0, keepdims=True).T


def _combstats(acc, den3, b_gat, batch2):
    return pl.pallas_call(
        _comb_body,
        grid=(GRID,),
        in_specs=[
            pl.BlockSpec((NC, R, H), lambda i: (0, i, 0)),
            pl.BlockSpec((NC, R, 1), lambda i: (0, i, 0)),
            pl.BlockSpec((1, H), lambda i: (0, 0)),
            pl.BlockSpec((R, 1), lambda i: (i, 0)),
        ],
        out_specs=[
            pl.BlockSpec((R, H), lambda i: (i, 0)),
            pl.BlockSpec((G, H), lambda i: (0, 0)),
            pl.BlockSpec((G, 1), lambda i: (0, 0)),
        ],
        out_shape=[
            jax.ShapeDtypeStruct((N, H), jnp.float32),
            jax.ShapeDtypeStruct((G, H), jnp.float32),
            jax.ShapeDtypeStruct((G, 1), jnp.float32),
        ],
    )(acc, den3, b_gat.reshape(1, H), batch2)


def _var_body(x_ref, s1_ref, cnt_ref, ms_ref, batch_ref, v_ref, mean_ref):
    i = pl.program_id(0)

    @pl.when(i == 0)
    def _():
        cnt = jnp.maximum(cnt_ref[...], 1.0)
        mean_ref[...] = s1_ref[...] / cnt * ms_ref[...]
        v_ref[...] = jnp.zeros_like(v_ref)

    oh = (batch_ref[...] == lax.broadcasted_iota(jnp.int32, (1, G), 1)
          ).astype(jnp.float32)
    mm = jnp.dot(oh, mean_ref[...], preferred_element_type=jnp.float32, precision=lax.Precision.HIGHEST)
    d = x_ref[...] - mm
    dn = (((0,), (0,)), ((), ()))
    v_ref[...] += lax.dot_general(oh, d * d, dn, preferred_element_type=jnp.float32, precision=lax.Precision.HIGHEST)


def _varstats(x, s1, cnt, gn_ms, batch2):
    return pl.pallas_call(
        _var_body,
        grid=(GRID,),
        in_specs=[
            pl.BlockSpec((R, H), lambda i: (i, 0)),
            pl.BlockSpec((G, H), lambda i: (0, 0)),
            pl.BlockSpec((G, 1), lambda i: (0, 0)),
            pl.BlockSpec((1, H), lambda i: (0, 0)),
            pl.BlockSpec((R, 1), lambda i: (i, 0)),
        ],
        out_specs=[
            pl.BlockSpec((G, H), lambda i: (0, 0)),
            pl.BlockSpec((G, H), lambda i: (0, 0)),
        ],
        out_shape=[
            jax.ShapeDtypeStruct((G, H), jnp.float32),
            jax.ShapeDtypeStruct((G, H), jnp.float32),
        ],
    )(x, s1, cnt, gn_ms.reshape(1, H), batch2)[0]


def _tail_body(x_ref, s1_ref, v_ref, cnt_ref, gw_ref, gb_ref, gms_ref,
               batch_ref, res_ref, h1_ref,
               wg1_ref, wg2_ref, bg_ref, wh_ref, bh_ref, g2_ref, b2_ref,
               wf1_ref, bf1_ref, g3_ref, b3_ref, wf2_ref, bf2_ref,
               out_ref, scale_ref, shift_ref):
    i = pl.program_id(0)
    y = _gn_block(x_ref, s1_ref, v_ref, cnt_ref, gw_ref, gb_ref, gms_ref,
                  batch_ref, scale_ref, shift_ref, i)
    h2 = y + res_ref[...]
    h1 = h1_ref[...]
    z = _bdot(h1, wg1_ref[...]) + _bdot(h2, wg2_ref[...]) + bg_ref[...]
    gate = _rcp(1.0 + jnp.exp(-z))
    h = gate * h1 + (1.0 - gate) * h2
    y = _bdot(h, wh_ref[...]) + bh_ref[...]
    y = jnp.maximum(_ln(y, g2_ref[...], b2_ref[...]), 0.0)
    y = _bdot(y, wf1_ref[...]) + bf1_ref[...]
    y = jnp.maximum(_ln(y, g3_ref[...], b3_ref[...]), 0.0)
    out_ref[...] = _bdot(y, wf2_ref[...]) + bf2_ref[...]


def _tail(x, s1, v, cnt, gn_w, gn_b, gn_ms, batch2, res, h1, W_gate, b_gate,
          W_head, b_head, ln2_g, ln2_b, W_fc1, b_fc1, ln3_g, ln3_b, W_fc2,
          b_fc2):
    H2 = H // 2
    return pl.pallas_call(
        _tail_body,
        grid=(GRID,),
        in_specs=[
            pl.BlockSpec((R, H), lambda i: (i, 0)),
            _W_FULL((G, H)), _W_FULL((G, H)), _W_FULL((G, 1)),
            _W_FULL((1, H)), _W_FULL((1, H)), _W_FULL((1, H)),
            pl.BlockSpec((R, 1), lambda i: (i, 0)),
            pl.BlockSpec((R, H), lambda i: (i, 0)),
            pl.BlockSpec((R, H), lambda i: (i, 0)),
            _W_FULL((H, H)), _W_FULL((H, H)), _W_FULL((1, H)),
            _W_FULL((H, H)), _W_FULL((1, H)),
            _W_FULL((1, H)), _W_FULL((1, H)),
            _W_FULL((H, H2)), _W_FULL((1, H2)),
            _W_FULL((1, H2)), _W_FULL((1, H2)),
            _W_FULL((H2, 1)), _W_FULL((1, 1)),
        ],
        out_specs=pl.BlockSpec((R, 1), lambda i: (i, 0)),
        out_shape=jax.ShapeDtypeStruct((N, 1), jnp.float32),
        scratch_shapes=[
            pltpu.VMEM((G, H), jnp.float32),
            pltpu.VMEM((G, H), jnp.float32),
        ],
    )(x, s1, v, cnt, gn_w.reshape(1, H), gn_b.reshape(1, H),
      gn_ms.reshape(1, H), batch2, res, h1,
      W_gate[:, :H].T, W_gate[:, H:].T, b_gate.reshape(1, H),
      W_head.T, b_head.reshape(1, H), ln2_g.reshape(1, H), ln2_b.reshape(1, H),
      W_fc1.T, b_fc1.reshape(1, H2), ln3_g.reshape(1, H2), ln3_b.reshape(1, H2),
      W_fc2.T, b_fc2.reshape(1, 1))


def kernel(rna_embeddings, ss_emb, W_fuse, b_fuse, ln1_g, ln1_b, W_gat,
           att_src, att_dst, b_gat, gn_w, gn_b, gn_ms, W_gate, b_gate, W_head,
           b_head, ln2_g, ln2_b, W_fc1, b_fc1, ln3_g, ln3_b, W_fc2, b_fc2,
           edge_index, batch):
    loop = jnp.arange(N, dtype=edge_index.dtype)
    srcp = jnp.pad(jnp.concatenate([edge_index[0], loop]), (0, EPAD + C - EL))
    dstp = jnp.pad(jnp.concatenate([edge_index[1], loop]), (0, EPAD + C - EL))
    batch2 = batch.reshape(N, 1)

    def edge_pass(hw, s, d, ms, md):
        m = ms[0, 0] + md[0, 0]
        m = jnp.where(m > 0.0, m, 0.2 * m)
        m16 = jnp.full((16,), m, jnp.float32)
        acc, den = _gat_agg(hw, s.reshape(N), d.reshape(N), srcp, dstp, m16)
        gat, s1, cnt = _combstats(acc, den.reshape(NC, NPAD, 1), b_gat, batch2)
        v = _varstats(gat, s1, cnt, gn_ms, batch2)
        return gat, s1, v, cnt

    h, hw1, s1a, d1a, ms1, md1 = _pre1(rna_embeddings, ss_emb, W_fuse, b_fuse,
                                       ln1_g, ln1_b, W_gat, att_src, att_dst)
    gat1, s1_1, v1, cnt1 = edge_pass(hw1, s1a, d1a, ms1, md1)
    h1, hw2, s2a, d2a, ms2, md2 = _mid(gat1, s1_1, v1, cnt1, gn_w, gn_b, gn_ms,
                                       batch2, W_gat, att_src, att_dst)
    gat2, s1_2, v2, cnt2 = edge_pass(hw2, s2a, d2a, ms2, md2)
    out = _tail(gat2, s1_2, v2, cnt2, gn_w, gn_b, gn_ms, batch2, h, h1,
                W_gate, b_gate, W_head, b_head, ln2_g, ln2_b, W_fc1, b_fc1,
                ln3_g, ln3_b, W_fc2, b_fc2)
    return out.reshape(N)
